# baseline (speedup 1.0000x reference)
import jax
import jax.numpy as jnp
from jax import lax
from jax.experimental import pallas as pl
from jax.experimental.pallas import tpu as pltpu
from jax.experimental.pallas import tpu_sc as plsc

_W = 512
_H = 512
_NBX = 512
_NBY = 512
_BIN_AREA = ((512.0 - 0.0) / _NBX) * ((512.0 - 0.0) / _NBY)

_NC = 1
_NS = 16
_L = 16
_NW = _NC * _NS
_RPW = _W // _NW
_CCH = _H // _L
_BAND = _RPW * _H
_PAD = 8


def _wcoef(n):
    w0 = jnp.maximum(jnp.minimum(n, 1.0), 0.0)
    w1 = jnp.maximum(jnp.minimum(n - 1.0, 1.0), 0.0)
    return w0, w1


def _sc_body(nsx_hbm, nsy_hbm, site_hbm, o1_hbm, o2_hbm, o3_hbm,
             site_v, ns_v, o1_v, o2_v, o3_v, sem0, sem1, sem2):
    wid = lax.axis_index("s")
    base = wid * _BAND

    half = _BAND // 2
    cx = pltpu.async_copy(nsx_hbm, ns_v.at[pl.ds(0, 4)], sem0)
    cy = pltpu.async_copy(nsy_hbm, ns_v.at[pl.ds(8, 4)], sem1)
    ch0 = pltpu.async_copy(site_hbm.at[pl.ds(base, half)],
                           site_v.at[pl.ds(_PAD + _H, half)], sem2)
    ch1 = pltpu.async_copy(site_hbm.at[pl.ds(base + half, half)],
                           site_v.at[pl.ds(_PAD + _H + half, half)], sem2)

    @pl.when(wid > 0)
    def _():
        pltpu.sync_copy(site_hbm.at[pl.ds(base - _H, _H)],
                        site_v.at[pl.ds(_PAD, _H)])

    @pl.when(wid == 0)
    def _():
        z = jnp.zeros((_L,), jnp.int32)

        def zero_chunk(i, c):
            site_v[pl.ds(_PAD + i * _L, _L)] = z
            return c

        lax.fori_loop(0, _CCH, zero_chunk, 0)

    cx.wait()
    cy.wait()

    def vgather(x, idx):
        return lax.gather(
            x, idx[:, None],
            lax.GatherDimensionNumbers(
                offset_dims=(), collapsed_slice_dims=(0,), start_index_map=(0,)),
            (1,), mode=lax.GatherScatterMode.PROMISE_IN_BOUNDS)

    ns = ns_v[pl.ds(0, _L)]
    zf = jnp.zeros((_L,), jnp.float32)
    coefs = []
    for lane in (0, 2, 3):
        w0, w1 = _wcoef(vgather(ns, jnp.full((_L,), lane, jnp.int32)))
        h0, h1 = _wcoef(vgather(ns, jnp.full((_L,), 8 + lane, jnp.int32)))
        coefs.append((w0 * h0, w1 * h0, w0 * h1, w1 * h1))

    iota = lax.broadcasted_iota(jnp.int32, (_L,), 0)
    outs = (o1_v, o2_v, o3_v)

    def process_rows(r0, nrows):
        def col_chunk(v, cc):
            cb = v * _L
            jmf = jnp.where((cb + iota) > 0, 1.0, 0.0)
            cj = [(c00, c10, c01 * jmf, c11 * jmf) for c00, c10, c01, c11 in coefs]

            def terms(off):
                s = site_v[pl.ds(_PAD + off, _L)]
                s_l = site_v[pl.ds(_PAD + off - 1, _L)]
                own, up = [], []
                for t in (1, 2, 3):
                    c00, c10, c01j, c11j = cj[t - 1]
                    e = s == t
                    el = s_l == t
                    own.append(jnp.where(e, c00, zf) + jnp.where(el, c01j, zf))
                    up.append(jnp.where(e, c10, zf) + jnp.where(el, c11j, zf))
                return own, up

            def row(r, carry):
                p_up = carry
                own, up = terms((r + 1) * _H + cb)
                for t in range(3):
                    outs[t][pl.ds(r * _H + cb, _L)] = _BIN_AREA - (own[t] + p_up[t])
                return tuple(up)

            _, up0 = terms(r0 * _H + cb)
            lax.fori_loop(r0, r0 + nrows, row, tuple(up0), unroll=2)
            return cc

        lax.fori_loop(0, _CCH, col_chunk, 0)

    ch0.wait()
    ch1.wait()
    process_rows(0, _RPW)
    pend = [pltpu.async_copy(o_v, o_hbm.at[pl.ds(base, _BAND)], sem0)
            for o_v, o_hbm in ((o1_v, o1_hbm), (o2_v, o2_hbm), (o3_v, o3_hbm))]
    for p in pend:
        p.wait()


def _sc_call(site_flat, nsx, nsy):
    out = jax.ShapeDtypeStruct((_NBX * _NBY,), jnp.float32)
    f = pl.kernel(
        _sc_body,
        mesh=plsc.VectorSubcoreMesh(core_axis_name="c", subcore_axis_name="s",
                                    num_cores=_NC),
        out_type=(out, out, out),
        scratch_types=[
            pltpu.VMEM((_PAD + (_RPW + 1) * _H,), jnp.int32),
            pltpu.VMEM((_L,), jnp.float32),
            pltpu.VMEM((_BAND,), jnp.float32),
            pltpu.VMEM((_BAND,), jnp.float32),
            pltpu.VMEM((_BAND,), jnp.float32),
            pltpu.SemaphoreType.DMA,
            pltpu.SemaphoreType.DMA,
            pltpu.SemaphoreType.DMA,
        ],
    )
    return f(nsx, nsy, site_flat)


def kernel(site_type_map, node_size_x, node_size_y):
    a, b, c = _sc_call(site_type_map.reshape(-1),
                       node_size_x.astype(jnp.float32),
                       node_size_y.astype(jnp.float32))
    a = a.reshape(_NBX, _NBY)
    b = b.reshape(_NBX, _NBY)
    c = c.reshape(_NBX, _NBY)
    return (a, a, a, a, a, b, c)


# speedup vs baseline: 1.1365x; 1.1365x over previous
import jax
import jax.numpy as jnp
from jax import lax
from jax.experimental import pallas as pl
from jax.experimental.pallas import tpu as pltpu
from jax.experimental.pallas import tpu_sc as plsc

_W = 512
_H = 512
_NBX = 512
_NBY = 512
_BIN_AREA = ((512.0 - 0.0) / _NBX) * ((512.0 - 0.0) / _NBY)

_NC = 2
_NS = 16
_L = 16
_NW = _NC * _NS
_RPW = _W // _NW
_CCH = _H // _L
_BAND = _RPW * _H
_PAD = 8


def _wcoef(n):
    w0 = jnp.maximum(jnp.minimum(n, 1.0), 0.0)
    w1 = jnp.maximum(jnp.minimum(n - 1.0, 1.0), 0.0)
    return w0, w1


def _sc_body(nsx_hbm, nsy_hbm, site_hbm, o1_hbm, o2_hbm, o3_hbm,
             site_v, ns_v, o1_v, o2_v, o3_v, sem0, sem1, sem2):
    wid = lax.axis_index("s") * _NC + lax.axis_index("c")
    base = wid * _BAND

    half = _BAND // 2
    cx = pltpu.async_copy(nsx_hbm, ns_v.at[pl.ds(0, 4)], sem0)
    cy = pltpu.async_copy(nsy_hbm, ns_v.at[pl.ds(8, 4)], sem1)
    ch0 = pltpu.async_copy(site_hbm.at[pl.ds(base, half)],
                           site_v.at[pl.ds(_PAD + _H, half)], sem2)
    ch1 = pltpu.async_copy(site_hbm.at[pl.ds(base + half, half)],
                           site_v.at[pl.ds(_PAD + _H + half, half)], sem2)

    @pl.when(wid > 0)
    def _():
        pltpu.sync_copy(site_hbm.at[pl.ds(base - _H, _H)],
                        site_v.at[pl.ds(_PAD, _H)])

    @pl.when(wid == 0)
    def _():
        z = jnp.zeros((_L,), jnp.int32)

        def zero_chunk(i, c):
            site_v[pl.ds(_PAD + i * _L, _L)] = z
            return c

        lax.fori_loop(0, _CCH, zero_chunk, 0)

    cx.wait()
    cy.wait()

    def vgather(x, idx):
        return lax.gather(
            x, idx[:, None],
            lax.GatherDimensionNumbers(
                offset_dims=(), collapsed_slice_dims=(0,), start_index_map=(0,)),
            (1,), mode=lax.GatherScatterMode.PROMISE_IN_BOUNDS)

    ns = ns_v[pl.ds(0, _L)]
    zf = jnp.zeros((_L,), jnp.float32)
    coefs = []
    for lane in (0, 2, 3):
        w0, w1 = _wcoef(vgather(ns, jnp.full((_L,), lane, jnp.int32)))
        h0, h1 = _wcoef(vgather(ns, jnp.full((_L,), 8 + lane, jnp.int32)))
        coefs.append((w0 * h0, w1 * h0, w0 * h1, w1 * h1))

    iota = lax.broadcasted_iota(jnp.int32, (_L,), 0)
    outs = (o1_v, o2_v, o3_v)

    def process_rows(r0, nrows):
        def col_chunk(v, cc):
            cb = v * _L
            jmf = jnp.where((cb + iota) > 0, 1.0, 0.0)
            cj = [(c00, c10, c01 * jmf, c11 * jmf) for c00, c10, c01, c11 in coefs]

            def terms(off):
                s = site_v[pl.ds(_PAD + off, _L)]
                s_l = site_v[pl.ds(_PAD + off - 1, _L)]
                own, up = [], []
                for t in (1, 2, 3):
                    c00, c10, c01j, c11j = cj[t - 1]
                    e = s == t
                    el = s_l == t
                    own.append(jnp.where(e, c00, zf) + jnp.where(el, c01j, zf))
                    up.append(jnp.where(e, c10, zf) + jnp.where(el, c11j, zf))
                return own, up

            def row(r, carry):
                p_up = carry
                own, up = terms((r + 1) * _H + cb)
                for t in range(3):
                    outs[t][pl.ds(r * _H + cb, _L)] = _BIN_AREA - (own[t] + p_up[t])
                return tuple(up)

            _, up0 = terms(r0 * _H + cb)
            lax.fori_loop(r0, r0 + nrows, row, tuple(up0), unroll=2)
            return cc

        lax.fori_loop(0, _CCH, col_chunk, 0)

    ch0.wait()
    ch1.wait()
    process_rows(0, _RPW)
    pend = [pltpu.async_copy(o_v, o_hbm.at[pl.ds(base, _BAND)], sem0)
            for o_v, o_hbm in ((o1_v, o1_hbm), (o2_v, o2_hbm), (o3_v, o3_hbm))]
    for p in pend:
        p.wait()


def _sc_call(site_flat, nsx, nsy):
    out = jax.ShapeDtypeStruct((_NBX * _NBY,), jnp.float32)
    f = pl.kernel(
        _sc_body,
        mesh=plsc.VectorSubcoreMesh(core_axis_name="c", subcore_axis_name="s",
                                    num_cores=_NC),
        out_type=(out, out, out),
        scratch_types=[
            pltpu.VMEM((_PAD + (_RPW + 1) * _H,), jnp.int32),
            pltpu.VMEM((_L,), jnp.float32),
            pltpu.VMEM((_BAND,), jnp.float32),
            pltpu.VMEM((_BAND,), jnp.float32),
            pltpu.VMEM((_BAND,), jnp.float32),
            pltpu.SemaphoreType.DMA,
            pltpu.SemaphoreType.DMA,
            pltpu.SemaphoreType.DMA,
        ],
    )
    return f(nsx, nsy, site_flat)


def kernel(site_type_map, node_size_x, node_size_y):
    a, b, c = _sc_call(site_type_map.reshape(-1),
                       node_size_x.astype(jnp.float32),
                       node_size_y.astype(jnp.float32))
    a = a.reshape(_NBX, _NBY)
    b = b.reshape(_NBX, _NBY)
    c = c.reshape(_NBX, _NBY)
    return (a, a, a, a, a, b, c)


# speedup vs baseline: 1.1392x; 1.0024x over previous
import jax
import jax.numpy as jnp
from jax import lax
from jax.experimental import pallas as pl
from jax.experimental.pallas import tpu as pltpu
from jax.experimental.pallas import tpu_sc as plsc

_W = 512
_H = 512
_NBX = 512
_NBY = 512
_BIN_AREA = ((512.0 - 0.0) / _NBX) * ((512.0 - 0.0) / _NBY)

_NC = 2
_NS = 16
_L = 16
_NW = _NC * _NS
_RPW = _W // _NW
_CCH = _H // _L
_BAND = _RPW * _H
_PAD = 8


def _wcoef(n):
    w0 = jnp.maximum(jnp.minimum(n, 1.0), 0.0)
    w1 = jnp.maximum(jnp.minimum(n - 1.0, 1.0), 0.0)
    return w0, w1


def _sc_body(nsx_hbm, nsy_hbm, site_hbm, o1_hbm, o2_hbm, o3_hbm,
             site_v, ns_v, o1_v, o2_v, o3_v, sem0, sem1, sem2):
    wid = lax.axis_index("s") * _NC + lax.axis_index("c")
    base = wid * _BAND

    half = _BAND // 2
    cx = pltpu.async_copy(nsx_hbm, ns_v.at[pl.ds(0, 4)], sem0)
    cy = pltpu.async_copy(nsy_hbm, ns_v.at[pl.ds(8, 4)], sem1)
    ch0 = pltpu.async_copy(site_hbm.at[pl.ds(base, half)],
                           site_v.at[pl.ds(_PAD + _H, half)], sem2)
    ch1 = pltpu.async_copy(site_hbm.at[pl.ds(base + half, half)],
                           site_v.at[pl.ds(_PAD + _H + half, half)], sem2)

    @pl.when(wid > 0)
    def _():
        pltpu.sync_copy(site_hbm.at[pl.ds(base - _H, _H)],
                        site_v.at[pl.ds(_PAD, _H)])

    @pl.when(wid == 0)
    def _():
        z = jnp.zeros((_L,), jnp.int32)

        def zero_chunk(i, c):
            site_v[pl.ds(_PAD + i * _L, _L)] = z
            return c

        lax.fori_loop(0, _CCH, zero_chunk, 0)

    cx.wait()
    cy.wait()

    def vgather(x, idx):
        return lax.gather(
            x, idx[:, None],
            lax.GatherDimensionNumbers(
                offset_dims=(), collapsed_slice_dims=(0,), start_index_map=(0,)),
            (1,), mode=lax.GatherScatterMode.PROMISE_IN_BOUNDS)

    ns = ns_v[pl.ds(0, _L)]
    zf = jnp.zeros((_L,), jnp.float32)
    coefs = []
    for lane in (0, 2, 3):
        w0, w1 = _wcoef(vgather(ns, jnp.full((_L,), lane, jnp.int32)))
        h0, h1 = _wcoef(vgather(ns, jnp.full((_L,), 8 + lane, jnp.int32)))
        coefs.append((w0 * h0, w1 * h0, w0 * h1, w1 * h1))

    iota = lax.broadcasted_iota(jnp.int32, (_L,), 0)
    outs = (o1_v, o2_v, o3_v)

    def process_rows(r0, nrows):
        @plsc.parallel_loop(0, _CCH, unroll=2)
        def col_chunk(v):
            cb = v * _L
            jmf = jnp.where((cb + iota) > 0, 1.0, 0.0)
            cj = [(c00, c10, c01 * jmf, c11 * jmf) for c00, c10, c01, c11 in coefs]

            def terms(off):
                s = site_v[pl.ds(_PAD + off, _L)]
                s_l = site_v[pl.ds(_PAD + off - 1, _L)]
                own, up = [], []
                for t in (1, 2, 3):
                    c00, c10, c01j, c11j = cj[t - 1]
                    e = s == t
                    el = s_l == t
                    own.append(jnp.where(e, c00, zf) + jnp.where(el, c01j, zf))
                    up.append(jnp.where(e, c10, zf) + jnp.where(el, c11j, zf))
                return own, up

            def row(r, carry):
                p_up = carry
                own, up = terms((r + 1) * _H + cb)
                for t in range(3):
                    outs[t][pl.ds(r * _H + cb, _L)] = _BIN_AREA - (own[t] + p_up[t])
                return tuple(up)

            _, up0 = terms(r0 * _H + cb)
            lax.fori_loop(r0, r0 + nrows, row, tuple(up0), unroll=2)

    ch0.wait()
    ch1.wait()
    process_rows(0, _RPW)
    pend = [pltpu.async_copy(o_v, o_hbm.at[pl.ds(base, _BAND)], sem0)
            for o_v, o_hbm in ((o1_v, o1_hbm), (o2_v, o2_hbm), (o3_v, o3_hbm))]
    for p in pend:
        p.wait()


def _sc_call(site_flat, nsx, nsy):
    out = jax.ShapeDtypeStruct((_NBX * _NBY,), jnp.float32)
    f = pl.kernel(
        _sc_body,
        mesh=plsc.VectorSubcoreMesh(core_axis_name="c", subcore_axis_name="s",
                                    num_cores=_NC),
        out_type=(out, out, out),
        scratch_types=[
            pltpu.VMEM((_PAD + (_RPW + 1) * _H,), jnp.int32),
            pltpu.VMEM((_L,), jnp.float32),
            pltpu.VMEM((_BAND,), jnp.float32),
            pltpu.VMEM((_BAND,), jnp.float32),
            pltpu.VMEM((_BAND,), jnp.float32),
            pltpu.SemaphoreType.DMA,
            pltpu.SemaphoreType.DMA,
            pltpu.SemaphoreType.DMA,
        ],
    )
    return f(nsx, nsy, site_flat)


def kernel(site_type_map, node_size_x, node_size_y):
    a, b, c = _sc_call(site_type_map.reshape(-1),
                       node_size_x.astype(jnp.float32),
                       node_size_y.astype(jnp.float32))
    a = a.reshape(_NBX, _NBY)
    b = b.reshape(_NBX, _NBY)
    c = c.reshape(_NBX, _NBY)
    return (a, a, a, a, a, b, c)
